# baseline (speedup 1.0000x reference)
import functools

import jax
import jax.numpy as jnp
from jax import lax
from jax.experimental import pallas as pl
from jax.experimental.pallas import tpu as pltpu
from jax.experimental.pallas import tpu_sc as plsc

D_MODEL = 128
VOCAB = 100000
_VB = 8192

_NC = 2
_NS = 16
_NW = _NC * _NS
_GB = 8


_NB = (VOCAB + _VB - 1) // _VB
_VTAIL = VOCAB - (_NB - 1) * _VB


def _transpose_body(w_ref, o_hbm, ot0, ot1, so0, so1):
    i = pl.program_id(0)
    even = i % 2 == 0
    last = _NB - 1

    def drain(ot, so, rows):
        pltpu.make_async_copy(
            ot.at[pl.ds(0, rows)], o_hbm.at[pl.ds(0, rows)], so).wait()

    def start_out(ot, so, rows):
        pltpu.make_async_copy(
            ot.at[pl.ds(0, rows)],
            o_hbm.at[pl.ds(i * _VB, rows)], so).start()

    @pl.when(jnp.logical_and(i >= 2, even))
    def _():
        drain(ot0, so0, _VB)

    @pl.when(jnp.logical_and(i >= 2, jnp.logical_not(even)))
    def _():
        drain(ot1, so1, _VB)

    @pl.when(even)
    def _():
        ot0[...] = w_ref[...].T

    @pl.when(jnp.logical_not(even))
    def _():
        ot1[...] = w_ref[...].T

    @pl.when(jnp.logical_and(i != last, even))
    def _():
        start_out(ot0, so0, _VB)

    @pl.when(jnp.logical_and(i != last, jnp.logical_not(even)))
    def _():
        start_out(ot1, so1, _VB)

    @pl.when(i == last)
    def _():
        ot, so = (ot0, so0) if last % 2 == 0 else (ot1, so1)
        po, ps = (ot1, so1) if last % 2 == 0 else (ot0, so0)
        start_out(ot, so, _VTAIL)
        drain(po, ps, _VB)
        drain(ot, so, _VTAIL)


def _transpose(W_E):
    return pl.pallas_call(
        _transpose_body,
        grid=(_NB,),
        in_specs=[pl.BlockSpec((D_MODEL, _VB), lambda i: (0, i))],
        out_specs=pl.BlockSpec(memory_space=pltpu.HBM),
        out_shape=jax.ShapeDtypeStruct((VOCAB, D_MODEL), jnp.float32),
        scratch_shapes=[
            pltpu.VMEM((_VB, D_MODEL), jnp.float32),
            pltpu.VMEM((_VB, D_MODEL), jnp.float32),
            pltpu.SemaphoreType.DMA,
            pltpu.SemaphoreType.DMA,
        ],
    )(W_E)


def _gather(table_t, idx3d, batch, n_ctx):
    per_w = idx3d.shape[1]
    n_groups = per_w // _GB
    mesh = plsc.VectorSubcoreMesh(core_axis_name="c", subcore_axis_name="s")

    @functools.partial(
        pl.kernel,
        mesh=mesh,
        out_type=jax.ShapeDtypeStruct((batch, n_ctx, D_MODEL), jnp.float32),
        scratch_types=[
            pltpu.VMEM((per_w, n_ctx), jnp.int32),
            pltpu.VMEM((_GB, n_ctx, D_MODEL), jnp.float32),
            pltpu.VMEM((_GB, n_ctx, D_MODEL), jnp.float32),
            pltpu.SemaphoreType.DMA,
            pltpu.SemaphoreType.DMA,
            pltpu.SemaphoreType.DMA,
        ],
        compiler_params=pltpu.CompilerParams(use_tc_tiling_on_sc=True),
    )
    def k(table_hbm, idx_hbm, out_hbm, idx_v, rows_a, rows_b, gsem, wsem_a,
          wsem_b):
        wid = lax.axis_index("s") * _NC + lax.axis_index("c")
        b0 = wid * per_w
        pltpu.sync_copy(idx_hbm.at[wid], idx_v)

        def do_group(g, rows_v, wsem):
            handles = [
                pltpu.async_copy(
                    table_hbm.at[idx_v.at[g * _GB + i]], rows_v.at[i], gsem)
                for i in range(_GB)
            ]
            for h in handles:
                h.wait()
            pltpu.async_copy(
                rows_v, out_hbm.at[pl.ds(b0 + g * _GB, _GB)], wsem)

        def drain_write(rows_v, wsem):
            pltpu.make_async_copy(
                rows_v, out_hbm.at[pl.ds(b0, _GB)], wsem).wait()

        def body(g, carry):
            even = g % 2 == 0

            @pl.when(jnp.logical_and(g >= 2, even))
            def _():
                drain_write(rows_a, wsem_a)

            @pl.when(jnp.logical_and(g >= 2, jnp.logical_not(even)))
            def _():
                drain_write(rows_b, wsem_b)

            @pl.when(even)
            def _():
                do_group(g, rows_a, wsem_a)

            @pl.when(jnp.logical_not(even))
            def _():
                do_group(g, rows_b, wsem_b)

            return carry

        lax.fori_loop(0, n_groups, body, 0)
        drain_write(rows_a, wsem_a)
        drain_write(rows_b, wsem_b)

    return k(table_t, idx3d)


def kernel(x, W_E):
    b, p = x.shape
    table_t = _transpose(W_E)
    idx3d = x.astype(jnp.int32).reshape(_NW, b // _NW, p)
    return _gather(table_t, idx3d, b, p)


# speedup vs baseline: 2.1699x; 2.1699x over previous
import functools

import jax
import jax.numpy as jnp
from jax import lax
from jax.experimental import pallas as pl
from jax.experimental.pallas import tpu as pltpu
from jax.experimental.pallas import tpu_sc as plsc

D_MODEL = 128
VOCAB = 100000

_NC = 2
_NS = 16
_NW = _NC * _NS


def _gather(table, idx_t, batch, n_ctx):
    per_w = batch // _NW
    mesh = plsc.VectorSubcoreMesh(core_axis_name="c", subcore_axis_name="s")

    @functools.partial(
        pl.kernel,
        mesh=mesh,
        out_type=jax.ShapeDtypeStruct((n_ctx, batch, D_MODEL), jnp.float32),
        scratch_types=[
            pltpu.VMEM((n_ctx, per_w), jnp.int32),
            pltpu.VMEM((per_w, D_MODEL), jnp.float32),
            pltpu.VMEM((per_w, D_MODEL), jnp.float32),
            pltpu.SemaphoreType.DMA,
            pltpu.SemaphoreType.DMA,
            pltpu.SemaphoreType.DMA,
        ],
    )
    def k(table_hbm, idx_hbm, out_hbm, idx_v, rows_a, rows_b, gsem, wsem_a,
          wsem_b):
        wid = lax.axis_index("s") * _NC + lax.axis_index("c")
        c0 = wid * per_w
        pltpu.sync_copy(idx_hbm.at[:, pl.ds(c0, per_w)], idx_v)

        def do_step(p, rows_v, wsem):
            pltpu.async_copy(table_hbm.at[idx_v.at[p]], rows_v, gsem).wait()
            pltpu.async_copy(rows_v, out_hbm.at[p, pl.ds(c0, per_w)], wsem)

        def drain_write(rows_v, wsem):
            pltpu.make_async_copy(
                rows_v, out_hbm.at[0, pl.ds(c0, per_w)], wsem).wait()

        def body(p, carry):
            even = p % 2 == 0

            @pl.when(jnp.logical_and(p >= 2, even))
            def _():
                drain_write(rows_a, wsem_a)

            @pl.when(jnp.logical_and(p >= 2, jnp.logical_not(even)))
            def _():
                drain_write(rows_b, wsem_b)

            @pl.when(even)
            def _():
                do_step(p, rows_a, wsem_a)

            @pl.when(jnp.logical_not(even))
            def _():
                do_step(p, rows_b, wsem_b)

            return carry

        lax.fori_loop(0, n_ctx, body, 0)
        drain_write(rows_a, wsem_a)
        drain_write(rows_b, wsem_b)

    return k(table, idx_t)


def kernel(x, W_E):
    b, p = x.shape
    table = jnp.swapaxes(W_E, 0, 1)
    idx_t = jnp.swapaxes(x, 0, 1).astype(jnp.int32)
    out_t = _gather(table, idx_t, b, p)
    return jnp.transpose(out_t, (1, 0, 2))


# speedup vs baseline: 2.6222x; 1.2084x over previous
import functools

import jax
import jax.numpy as jnp
from jax import lax
from jax.experimental import pallas as pl
from jax.experimental.pallas import tpu as pltpu
from jax.experimental.pallas import tpu_sc as plsc

D_MODEL = 128
VOCAB = 100000

_NC = 2
_NS = 16
_NW = _NC * _NS
_NBUF = 5


def _gather(table, idx_t, batch, n_ctx):
    per_w = batch // _NW
    mesh = plsc.VectorSubcoreMesh(core_axis_name="c", subcore_axis_name="s")

    @functools.partial(
        pl.kernel,
        mesh=mesh,
        out_type=jax.ShapeDtypeStruct((n_ctx, batch, D_MODEL), jnp.float32),
        scratch_types=(
            [pltpu.VMEM((n_ctx, per_w), jnp.int32)]
            + [pltpu.VMEM((per_w, D_MODEL), jnp.float32)] * _NBUF
            + [pltpu.SemaphoreType.DMA] * (2 * _NBUF)
        ),
    )
    def k(table_hbm, idx_hbm, out_hbm, idx_v, *bufs_and_sems):
        rows = bufs_and_sems[:_NBUF]
        gsems = bufs_and_sems[_NBUF:2 * _NBUF]
        wsems = bufs_and_sems[2 * _NBUF:]
        wid = lax.axis_index("s") * _NC + lax.axis_index("c")
        c0 = wid * per_w
        pltpu.sync_copy(idx_hbm.at[:, pl.ds(c0, per_w)], idx_v)

        def drain_write(i):
            pltpu.make_async_copy(
                rows[i], out_hbm.at[0, pl.ds(c0, per_w)], wsems[i]).wait()

        rounds = n_ctx // _NBUF

        def body(r, carry):
            for i in range(_NBUF):
                p = r * _NBUF + i

                @pl.when(r >= 1)
                def _(i=i):
                    drain_write(i)

                pltpu.async_copy(table_hbm.at[idx_v.at[p]], rows[i],
                                 gsems[i])
            for i in range(_NBUF):
                p = r * _NBUF + i
                pltpu.make_async_copy(
                    table_hbm.at[idx_v.at[p]], rows[i], gsems[i]).wait()
                pltpu.async_copy(rows[i], out_hbm.at[p, pl.ds(c0, per_w)],
                                 wsems[i])
            return carry

        lax.fori_loop(0, rounds, body, 0)
        for i in range(_NBUF):
            drain_write(i)

    return k(table, idx_t)


def kernel(x, W_E):
    b, p = x.shape
    table = jnp.swapaxes(W_E, 0, 1)
    idx_t = jnp.swapaxes(x, 0, 1).astype(jnp.int32)
    out_t = _gather(table, idx_t, b, p)
    return jnp.transpose(out_t, (1, 0, 2))


# speedup vs baseline: 2.6787x; 1.0216x over previous
import functools

import jax
import jax.numpy as jnp
from jax import lax
from jax.experimental import pallas as pl
from jax.experimental.pallas import tpu as pltpu
from jax.experimental.pallas import tpu_sc as plsc

D_MODEL = 128
VOCAB = 100000

_NC = 2
_NS = 16
_NW = _NC * _NS
_NBUF = 10
_CH = 64


def _gather(table, idx_t, batch, n_ctx):
    per_w = batch // _NW
    mesh = plsc.VectorSubcoreMesh(core_axis_name="c", subcore_axis_name="s")

    @functools.partial(
        pl.kernel,
        mesh=mesh,
        out_type=jax.ShapeDtypeStruct((n_ctx, batch, D_MODEL), jnp.float32),
        scratch_types=(
            [pltpu.VMEM((n_ctx, per_w), jnp.int32)]
            + [pltpu.VMEM((_CH, D_MODEL), jnp.float32)] * _NBUF
            + [pltpu.SemaphoreType.DMA] * (2 * _NBUF)
        ),
    )
    def k(table_hbm, idx_hbm, out_hbm, idx_v, *bufs_and_sems):
        rows = bufs_and_sems[:_NBUF]
        gsems = bufs_and_sems[_NBUF:2 * _NBUF]
        wsems = bufs_and_sems[2 * _NBUF:]
        wid = lax.axis_index("s") * _NC + lax.axis_index("c")
        c0 = wid * per_w
        pltpu.sync_copy(idx_hbm.at[:, pl.ds(c0, per_w)], idx_v)

        def drain_write(i):
            pltpu.make_async_copy(
                rows[i], out_hbm.at[0, pl.ds(c0, _CH)], wsems[i]).wait()

        steps_per_col = per_w // _CH
        rounds = n_ctx * steps_per_col // _NBUF

        def body(r, carry):
            for i in range(_NBUF):
                s = r * _NBUF + i
                p, h = s // steps_per_col, s % steps_per_col

                @pl.when(r >= 1)
                def _(i=i):
                    drain_write(i)

                pltpu.async_copy(
                    table_hbm.at[idx_v.at[p, pl.ds(h * _CH, _CH)]], rows[i],
                    gsems[i])
            for i in range(_NBUF):
                s = r * _NBUF + i
                p, h = s // steps_per_col, s % steps_per_col
                pltpu.make_async_copy(
                    table_hbm.at[idx_v.at[p, pl.ds(h * _CH, _CH)]], rows[i],
                    gsems[i]).wait()
                pltpu.async_copy(
                    rows[i], out_hbm.at[p, pl.ds(c0 + h * _CH, _CH)],
                    wsems[i])
            return carry

        lax.fori_loop(0, rounds, body, 0)
        for i in range(_NBUF):
            drain_write(i)

    return k(table, idx_t)


def kernel(x, W_E):
    b, p = x.shape
    table = jnp.swapaxes(W_E, 0, 1)
    idx_t = jnp.swapaxes(x, 0, 1).astype(jnp.int32)
    out_t = _gather(table, idx_t, b, p)
    return jnp.transpose(out_t, (1, 0, 2))
